# baseline (speedup 1.0000x reference)
import functools

import jax
import jax.numpy as jnp
from jax.experimental import pallas as pl
from jax.experimental.pallas import tpu as pltpu


def _enc_kernel(x_ref, adjt_ref, adjb_ref, w1_ref, w23_ref, wa1_ref,
                wa2_ref, wa3_ref,
                mut_ref, mub_ref, lvt_ref, lvb_ref, mua_ref, logvara_ref,
                p_sc, hw_sc, *, S, bm, H2, Nh):
    i = pl.program_id(0)

    @pl.when(i == 0)
    def _prep():
        xv = x_ref[...]
        p_sc[...] = jnp.dot(xv, w1_ref[...], preferred_element_type=jnp.float32)
        ha1 = jnp.tanh(jax.lax.dot_general(
            xv, wa1_ref[...], (((0,), (0,)), ((), ())),
            preferred_element_type=jnp.float32))
        mua_ref[...] = jnp.dot(ha1, wa2_ref[...],
                               preferred_element_type=jnp.float32)
        logvara_ref[...] = jnp.dot(ha1, wa3_ref[...],
                                   preferred_element_type=jnp.float32)

    @pl.when(i < S)
    def _phase_a():
        ht = jnp.maximum(
            jnp.dot(adjt_ref[...], p_sc[...],
                    preferred_element_type=jnp.float32), 0.0)
        hw_sc[pl.ds(i * bm, bm), :] = jnp.dot(
            ht, w23_ref[...], preferred_element_type=jnp.float32)
        hb = jnp.maximum(
            jnp.dot(adjb_ref[...], p_sc[...],
                    preferred_element_type=jnp.float32), 0.0)
        hw_sc[pl.ds(Nh + i * bm, bm), :] = jnp.dot(
            hb, w23_ref[...], preferred_element_type=jnp.float32)

    @pl.when(i >= S)
    def _phase_b():
        mlt = jnp.dot(adjt_ref[...], hw_sc[...],
                      preferred_element_type=jnp.float32)
        mut_ref[...] = mlt[:, :H2]
        lvt_ref[...] = mlt[:, H2:]
        mlb = jnp.dot(adjb_ref[...], hw_sc[...],
                      preferred_element_type=jnp.float32)
        mub_ref[...] = mlb[:, :H2]
        lvb_ref[...] = mlb[:, H2:]


def _dec_kernel(mu_blk_ref, mu_full_ref, mua_ref, adjrec_ref, xrec_ref):
    mu_i = mu_blk_ref[...]
    adjrec_ref[...] = jax.lax.dot_general(
        mu_i, mu_full_ref[...], (((1,), (1,)), ((), ())),
        preferred_element_type=jnp.float32)
    xrec_ref[...] = jax.lax.dot_general(
        mu_i, mua_ref[...], (((1,), (1,)), ((), ())),
        preferred_element_type=jnp.float32)


def kernel(x, adj, W1, W2, W3, Wa1, Wa2, Wa3):
    N, D = x.shape
    H1 = W1.shape[1]
    H2 = W2.shape[1]
    Nh = N // 2
    bm = 200 if Nh % 200 == 0 else 8
    S = Nh // bm

    W23 = jnp.concatenate([W2, W3], axis=1)

    zero2 = lambda i: (0, 0)

    def top_idx(i):
        return (jax.lax.rem(i, S), 0)

    def bot_idx(i):
        return (S + jax.lax.rem(i, S), 0)

    def half_out_idx(i):
        return (jnp.clip(i - S, 0, S - 1), 0)

    mu_t, mu_b, lv_t, lv_b, mu_a, logvar_a = pl.pallas_call(
        functools.partial(_enc_kernel, S=S, bm=bm, H2=H2, Nh=Nh),
        grid=(2 * S,),
        in_specs=[
            pl.BlockSpec((N, D), zero2),
            pl.BlockSpec((bm, N), top_idx),
            pl.BlockSpec((bm, N), bot_idx),
            pl.BlockSpec((D, H1), zero2),
            pl.BlockSpec((H1, 2 * H2), zero2),
            pl.BlockSpec((N, H1), zero2),
            pl.BlockSpec((H1, H2), zero2),
            pl.BlockSpec((H1, H2), zero2),
        ],
        out_specs=(
            pl.BlockSpec((bm, H2), half_out_idx),
            pl.BlockSpec((bm, H2), half_out_idx),
            pl.BlockSpec((bm, H2), half_out_idx),
            pl.BlockSpec((bm, H2), half_out_idx),
            pl.BlockSpec((D, H2), zero2),
            pl.BlockSpec((D, H2), zero2),
        ),
        out_shape=(
            jax.ShapeDtypeStruct((Nh, H2), jnp.float32),
            jax.ShapeDtypeStruct((Nh, H2), jnp.float32),
            jax.ShapeDtypeStruct((Nh, H2), jnp.float32),
            jax.ShapeDtypeStruct((Nh, H2), jnp.float32),
            jax.ShapeDtypeStruct((D, H2), jnp.float32),
            jax.ShapeDtypeStruct((D, H2), jnp.float32),
        ),
        scratch_shapes=[
            pltpu.VMEM((N, H1), jnp.float32),
            pltpu.VMEM((N, 2 * H2), jnp.float32),
        ],
        compiler_params=pltpu.CompilerParams(
            dimension_semantics=("arbitrary",),
            vmem_limit_bytes=63 * 1024 * 1024),
    )(x, adj, adj, W1, W23, Wa1, Wa2, Wa3)

    mu = jnp.concatenate([mu_t, mu_b], axis=0)
    logvar = jnp.concatenate([lv_t, lv_b], axis=0)

    bd = 400 if N % 400 == 0 else 8
    Sd = N // bd

    adj_rec, x_rec = pl.pallas_call(
        _dec_kernel,
        grid=(Sd,),
        in_specs=[
            pl.BlockSpec((bd, H2), lambda i: (i, 0)),
            pl.BlockSpec((N, H2), zero2),
            pl.BlockSpec((D, H2), zero2),
        ],
        out_specs=(
            pl.BlockSpec((bd, N), lambda i: (i, 0)),
            pl.BlockSpec((bd, D), lambda i: (i, 0)),
        ),
        out_shape=(
            jax.ShapeDtypeStruct((N, N), jnp.float32),
            jax.ShapeDtypeStruct((N, D), jnp.float32),
        ),
        compiler_params=pltpu.CompilerParams(
            dimension_semantics=("parallel",)),
    )(mu, mu, mu_a)

    return (adj_rec, x_rec, mu, logvar, mu_a, logvar_a)
